# baseline (speedup 1.0000x reference)
import functools

import jax
import jax.numpy as jnp
from jax import lax
from jax.experimental import pallas as pl
from jax.experimental.pallas import tpu as pltpu
from jax.experimental.pallas import tpu_sc as plsc

C = 4096
L = 16
NCHUNK = C // L
NC = 2
NS = 16
NW = NC * NS
RPW = C // NW
LABEL_SMOOTH = 0.1
EPS = LABEL_SMOOTH / (C - 1)
HI = 1.0 - LABEL_SMOOTH


GRP = 16
NGRP = 256 // GRP


def _sc_body(preds_hbm, tgt_hbm, stats_hbm, cval_hbm, cidx_hbm,
             rowbuf0, rowbuf1, tgtbuf, gbuf, trv, tri,
             stats_v, cval_v, cidx_v, sem0, sem1):
    wid = lax.axis_index("s") * NC + lax.axis_index("c")
    base = wid * RPW
    pltpu.sync_copy(tgt_hbm.at[pl.ds(base, RPW)], tgtbuf)

    iota = lax.iota(jnp.int32, L)
    zf = jnp.zeros((L,), jnp.float32)
    zi = jnp.zeros((L,), jnp.int32)
    ninf = jnp.full((L,), -jnp.inf, jnp.float32)

    bufs = (rowbuf0, rowbuf1)
    sems = (sem0, sem1)
    pltpu.async_copy(preds_hbm.at[pl.ds(base * C, C)], rowbuf0, sem0)

    def do_row(j, rowbuf, sem, osem, obuf):
        row = base + j
        pltpu.make_async_copy(preds_hbm.at[pl.ds(base * C, C)], rowbuf,
                              sem).wait()

        @pl.when(j + 1 < RPW)
        def _():
            pltpu.async_copy(preds_hbm.at[pl.ds((row + 1) * C, C)], obuf,
                             osem)

        def ga(g, carry):
            m_v, s_v = carry

            def gi(k, carry2):
                gm, s2 = carry2
                v = rowbuf[pl.ds((g * GRP + k) * L, L)]
                return (jnp.maximum(gm, v), s2 + v)

            gm, s_v = lax.fori_loop(0, GRP, gi, (ninf, s_v), unroll=8)
            gbuf[pl.ds(g * L, L)] = gm
            return (jnp.maximum(m_v, gm), s_v)

        m_v, s_v = lax.fori_loop(0, NGRP, ga, (ninf, zf))

        mm = m_v
        for _ in range(3):
            t = jnp.max(mm)
            mm = jnp.where(mm == jnp.full((L,), t), ninf, mm)
        tau_v = jnp.full((L,), jnp.max(mm))

        trv[pl.ds(0, L)] = ninf
        trv[pl.ds(16, L)] = ninf
        trv[pl.ds(32, L)] = ninf
        trv[pl.ds(48, L)] = ninf
        tri[pl.ds(0, L)] = zi
        tri[pl.ds(16, L)] = zi
        tri[pl.ds(32, L)] = zi
        tri[pl.ds(48, L)] = zi

        def gb(g, _):
            gm = gbuf[pl.ds(g * L, L)]
            hit = jnp.any(gm >= tau_v)

            @pl.when(hit)
            def _():
                r0v = trv[pl.ds(0, L)]
                r1v = trv[pl.ds(16, L)]
                r2v = trv[pl.ds(32, L)]
                r3v = trv[pl.ds(48, L)]
                r0i = tri[pl.ds(0, L)]
                r1i = tri[pl.ds(16, L)]
                r2i = tri[pl.ds(32, L)]
                r3i = tri[pl.ds(48, L)]

                def ins(k, carry):
                    c0v, c1v, c2v, c3v, c0i, c1i, c2i, c3i = carry
                    c = g * GRP + k
                    v = rowbuf[pl.ds(c * L, L)]
                    cols = iota + c * L
                    w0 = v > c0v
                    w1 = v > c1v
                    w2 = v > c2v
                    w3 = v > c3v
                    n0v = jnp.where(w0, v, c0v)
                    n0i = jnp.where(w0, cols, c0i)
                    n1v = jnp.where(w0, c0v, jnp.where(w1, v, c1v))
                    n1i = jnp.where(w0, c0i, jnp.where(w1, cols, c1i))
                    n2v = jnp.where(w1, c1v, jnp.where(w2, v, c2v))
                    n2i = jnp.where(w1, c1i, jnp.where(w2, cols, c2i))
                    n3v = jnp.where(w2, c2v, jnp.where(w3, v, c3v))
                    n3i = jnp.where(w2, c2i, jnp.where(w3, cols, c3i))
                    return (n0v, n1v, n2v, n3v, n0i, n1i, n2i, n3i)

                r0v, r1v, r2v, r3v, r0i, r1i, r2i, r3i = lax.fori_loop(
                    0, GRP, ins,
                    (r0v, r1v, r2v, r3v, r0i, r1i, r2i, r3i), unroll=4)
                trv[pl.ds(0, L)] = r0v
                trv[pl.ds(16, L)] = r1v
                trv[pl.ds(32, L)] = r2v
                trv[pl.ds(48, L)] = r3v
                tri[pl.ds(0, L)] = r0i
                tri[pl.ds(16, L)] = r1i
                tri[pl.ds(32, L)] = r2i
                tri[pl.ds(48, L)] = r3i

            return 0

        lax.fori_loop(0, NGRP, gb, 0)

        def p2(c, e):
            v = rowbuf[pl.ds(c * L, L)]
            return e + jnp.exp(v - m_v)

        e_v = lax.fori_loop(0, NCHUNK, p2, zf, unroll=8)

        tj = plsc.load_gather(tgtbuf, [jnp.full((L,), j, jnp.int32)])
        ptv = plsc.load_gather(rowbuf, [tj])

        sb = j * 64
        stats_v[pl.ds(sb, L)] = m_v
        stats_v[pl.ds(sb + 16, L)] = s_v
        stats_v[pl.ds(sb + 32, L)] = e_v
        stats_v[pl.ds(sb + 48, L)] = ptv
        cval_v[pl.ds(sb, L)] = trv[pl.ds(0, L)]
        cval_v[pl.ds(sb + 16, L)] = trv[pl.ds(16, L)]
        cval_v[pl.ds(sb + 32, L)] = trv[pl.ds(32, L)]
        cval_v[pl.ds(sb + 48, L)] = trv[pl.ds(48, L)]
        cidx_v[pl.ds(sb, L)] = tri[pl.ds(0, L)]
        cidx_v[pl.ds(sb + 16, L)] = tri[pl.ds(16, L)]
        cidx_v[pl.ds(sb + 32, L)] = tri[pl.ds(32, L)]
        cidx_v[pl.ds(sb + 48, L)] = tri[pl.ds(48, L)]

    def pair_step(g, _):
        do_row(2 * g, bufs[0], sems[0], sems[1], bufs[1])
        do_row(2 * g + 1, bufs[1], sems[1], sems[0], bufs[0])
        return 0

    lax.fori_loop(0, RPW // 2, pair_step, 0)

    pltpu.sync_copy(stats_v, stats_hbm.at[pl.ds(base * 64, RPW * 64)])
    pltpu.sync_copy(cval_v, cval_hbm.at[pl.ds(base * 64, RPW * 64)])
    pltpu.sync_copy(cidx_v, cidx_hbm.at[pl.ds(base * 64, RPW * 64)])


def _fin_body(stats_ref, cval_ref, cidx_ref, tgt_ref, out_ref):
    i = pl.program_id(0)
    R = stats_ref.shape[0]
    stats = stats_ref[...]
    m_v = stats[:, 0:16]
    sum_v = stats[:, 16:32]
    e_v = stats[:, 32:48]
    pt = jnp.max(stats[:, 48:64], axis=1)

    M = jnp.max(m_v, axis=1)
    S = jnp.sum(e_v * jnp.exp(m_v - M[:, None]), axis=1)
    lse = M + jnp.log(S)
    rowsum = jnp.sum(sum_v, axis=1)
    full = EPS * (rowsum - C * lse) + (HI - EPS) * (pt - lse)

    cval = cval_ref[...]
    cidx = cidx_ref[...]
    alive = jnp.ones(cval.shape, jnp.bool_)
    tv = []
    ti = []
    for _ in range(4):
        mv = jnp.where(alive, cval, -jnp.inf)
        cur = jnp.max(mv, axis=1)
        cand = mv == cur[:, None]
        curi = jnp.min(jnp.where(cand, cidx, C), axis=1)
        tv.append(cur)
        ti.append(curi)
        alive = alive & ~(cand & (cidx == curi[:, None]))

    rows = i * R + lax.broadcasted_iota(jnp.int32, (R,), 0)
    in0 = ti[0] == rows
    in1 = ti[1] == rows
    in2 = ti[2] == rows
    tgt = tgt_ref[:, 0]

    def term(v, idx):
        w = jnp.where(idx == tgt, HI, EPS)
        return w * (v - lse)

    sk0 = jnp.where(in0, term(tv[1], ti[1]), term(tv[0], ti[0]))
    sk1 = jnp.where(in0 | in1, term(tv[2], ti[2]), term(tv[1], ti[1]))
    sk2 = jnp.where(in0 | in1 | in2, term(tv[3], ti[3]), term(tv[2], ti[2]))
    skipped = sk0 + sk1 + sk2

    loss = -(full - skipped)
    part = jnp.reshape(jnp.sum(loss) * (1.0 / C), (1, 1))

    @pl.when(i == 0)
    def _():
        out_ref[...] = jnp.zeros((1, 1), jnp.float32)

    out_ref[...] += part


def _sc_call(preds_flat, targets):
    mesh = plsc.VectorSubcoreMesh(core_axis_name="c", subcore_axis_name="s",
                                  num_cores=NC, num_subcores=NS)
    f = functools.partial(
        pl.kernel,
        mesh=mesh,
        out_type=[
            jax.ShapeDtypeStruct((C * 64,), jnp.float32),
            jax.ShapeDtypeStruct((C * 64,), jnp.float32),
            jax.ShapeDtypeStruct((C * 64,), jnp.int32),
        ],
        scratch_types=[
            pltpu.VMEM((C,), jnp.float32),
            pltpu.VMEM((C,), jnp.float32),
            pltpu.VMEM((RPW,), jnp.int32),
            pltpu.VMEM((NGRP * L,), jnp.float32),
            pltpu.VMEM((64,), jnp.float32),
            pltpu.VMEM((64,), jnp.int32),
            pltpu.VMEM((RPW * 64,), jnp.float32),
            pltpu.VMEM((RPW * 64,), jnp.float32),
            pltpu.VMEM((RPW * 64,), jnp.int32),
            pltpu.SemaphoreType.DMA,
            pltpu.SemaphoreType.DMA,
        ],
        compiler_params=pltpu.CompilerParams(needs_layout_passes=False),
    )(_sc_body)
    return f(preds_flat, targets)


def kernel(preds, targets):
    preds_flat = preds.reshape(-1)
    tgt = targets.astype(jnp.int32)
    stats, cval, cidx = _sc_call(preds_flat, tgt)

    R = 512
    out = pl.pallas_call(
        _fin_body,
        grid=(C // R,),
        in_specs=[
            pl.BlockSpec((R, 64), lambda i: (i, 0)),
            pl.BlockSpec((R, 64), lambda i: (i, 0)),
            pl.BlockSpec((R, 64), lambda i: (i, 0)),
            pl.BlockSpec((R, 1), lambda i: (i, 0)),
        ],
        out_specs=pl.BlockSpec((1, 1), lambda i: (0, 0)),
        out_shape=jax.ShapeDtypeStruct((1, 1), jnp.float32),
    )(stats.reshape(C, 64), cval.reshape(C, 64), cidx.reshape(C, 64),
      tgt.reshape(C, 1))
    return out[0, 0]


# speedup vs baseline: 1.0785x; 1.0785x over previous
import functools

import jax
import jax.numpy as jnp
from jax import lax
from jax.experimental import pallas as pl
from jax.experimental.pallas import tpu as pltpu
from jax.experimental.pallas import tpu_sc as plsc

C = 4096
L = 16
NCHUNK = C // L
NC = 2
NS = 16
NW = NC * NS
RPW = C // NW
LABEL_SMOOTH = 0.1
EPS = LABEL_SMOOTH / (C - 1)
HI = 1.0 - LABEL_SMOOTH


GRP = 16
NGRP = 256 // GRP


def _sc_body(preds_hbm, tgt_hbm, stats_hbm, cval_hbm, cidx_hbm,
             rowbuf0, rowbuf1, tgtbuf, gbuf, trv, tri,
             stats_v, cval_v, cidx_v, sem0, sem1):
    wid = lax.axis_index("s") * NC + lax.axis_index("c")
    base = wid * RPW
    pltpu.sync_copy(tgt_hbm.at[pl.ds(base, RPW)], tgtbuf)

    iota = lax.iota(jnp.int32, L)
    zf = jnp.zeros((L,), jnp.float32)
    zi = jnp.zeros((L,), jnp.int32)
    ninf = jnp.full((L,), -jnp.inf, jnp.float32)

    bufs = (rowbuf0, rowbuf1)
    sems = (sem0, sem1)
    pltpu.async_copy(preds_hbm.at[pl.ds(base * C, C)], rowbuf0, sem0)

    def do_row(j, rowbuf, sem, osem, obuf):
        row = base + j
        pltpu.make_async_copy(preds_hbm.at[pl.ds(base * C, C)], rowbuf,
                              sem).wait()

        @pl.when(j + 1 < RPW)
        def _():
            pltpu.async_copy(preds_hbm.at[pl.ds((row + 1) * C, C)], obuf,
                             osem)

        def ga(g, carry):
            m_v, s0, s1, s2, s3 = carry

            def gi(t, c2):
                a0, a1, a2, a3, s0, s1, s2, s3 = c2
                b = (g * GRP + t * 4) * L
                v0 = rowbuf[pl.ds(b, L)]
                v1 = rowbuf[pl.ds(b + L, L)]
                v2 = rowbuf[pl.ds(b + 2 * L, L)]
                v3 = rowbuf[pl.ds(b + 3 * L, L)]
                return (jnp.maximum(a0, v0), jnp.maximum(a1, v1),
                        jnp.maximum(a2, v2), jnp.maximum(a3, v3),
                        s0 + v0, s1 + v1, s2 + v2, s3 + v3)

            a0, a1, a2, a3, s0, s1, s2, s3 = lax.fori_loop(
                0, GRP // 4, gi,
                (ninf, ninf, ninf, ninf, s0, s1, s2, s3), unroll=4)
            gm = jnp.maximum(jnp.maximum(a0, a1), jnp.maximum(a2, a3))
            gbuf[pl.ds(g * L, L)] = gm
            return (jnp.maximum(m_v, gm), s0, s1, s2, s3)

        m_v, s0, s1, s2, s3 = lax.fori_loop(
            0, NGRP, ga, (ninf, zf, zf, zf, zf))
        s_v = (s0 + s1) + (s2 + s3)

        mm = m_v
        for _ in range(3):
            t = jnp.max(mm)
            mm = jnp.where(mm == jnp.full((L,), t), ninf, mm)
        tau_v = jnp.full((L,), jnp.max(mm))

        trv[pl.ds(0, L)] = ninf
        trv[pl.ds(16, L)] = ninf
        trv[pl.ds(32, L)] = ninf
        trv[pl.ds(48, L)] = ninf
        tri[pl.ds(0, L)] = zi
        tri[pl.ds(16, L)] = zi
        tri[pl.ds(32, L)] = zi
        tri[pl.ds(48, L)] = zi

        def gb(g, _):
            gm = gbuf[pl.ds(g * L, L)]
            hit = jnp.any(gm >= tau_v)

            @pl.when(hit)
            def _():
                r0v = trv[pl.ds(0, L)]
                r1v = trv[pl.ds(16, L)]
                r2v = trv[pl.ds(32, L)]
                r3v = trv[pl.ds(48, L)]
                r0i = tri[pl.ds(0, L)]
                r1i = tri[pl.ds(16, L)]
                r2i = tri[pl.ds(32, L)]
                r3i = tri[pl.ds(48, L)]

                def ins(k, carry):
                    c0v, c1v, c2v, c3v, c0i, c1i, c2i, c3i = carry
                    c = g * GRP + k
                    v = rowbuf[pl.ds(c * L, L)]
                    cols = iota + c * L
                    w0 = v > c0v
                    w1 = v > c1v
                    w2 = v > c2v
                    w3 = v > c3v
                    n0v = jnp.where(w0, v, c0v)
                    n0i = jnp.where(w0, cols, c0i)
                    n1v = jnp.where(w0, c0v, jnp.where(w1, v, c1v))
                    n1i = jnp.where(w0, c0i, jnp.where(w1, cols, c1i))
                    n2v = jnp.where(w1, c1v, jnp.where(w2, v, c2v))
                    n2i = jnp.where(w1, c1i, jnp.where(w2, cols, c2i))
                    n3v = jnp.where(w2, c2v, jnp.where(w3, v, c3v))
                    n3i = jnp.where(w2, c2i, jnp.where(w3, cols, c3i))
                    return (n0v, n1v, n2v, n3v, n0i, n1i, n2i, n3i)

                r0v, r1v, r2v, r3v, r0i, r1i, r2i, r3i = lax.fori_loop(
                    0, GRP, ins,
                    (r0v, r1v, r2v, r3v, r0i, r1i, r2i, r3i), unroll=4)
                trv[pl.ds(0, L)] = r0v
                trv[pl.ds(16, L)] = r1v
                trv[pl.ds(32, L)] = r2v
                trv[pl.ds(48, L)] = r3v
                tri[pl.ds(0, L)] = r0i
                tri[pl.ds(16, L)] = r1i
                tri[pl.ds(32, L)] = r2i
                tri[pl.ds(48, L)] = r3i

            return 0

        lax.fori_loop(0, NGRP, gb, 0)

        def p2(t, c2):
            e0, e1, e2, e3 = c2
            b = t * 4 * L
            e0 = e0 + jnp.exp(rowbuf[pl.ds(b, L)] - m_v)
            e1 = e1 + jnp.exp(rowbuf[pl.ds(b + L, L)] - m_v)
            e2 = e2 + jnp.exp(rowbuf[pl.ds(b + 2 * L, L)] - m_v)
            e3 = e3 + jnp.exp(rowbuf[pl.ds(b + 3 * L, L)] - m_v)
            return (e0, e1, e2, e3)

        e0, e1, e2, e3 = lax.fori_loop(0, NCHUNK // 4, p2,
                                       (zf, zf, zf, zf), unroll=4)
        e_v = (e0 + e1) + (e2 + e3)

        tj = plsc.load_gather(tgtbuf, [jnp.full((L,), j, jnp.int32)])
        ptv = plsc.load_gather(rowbuf, [tj])

        sb = j * 64
        stats_v[pl.ds(sb, L)] = m_v
        stats_v[pl.ds(sb + 16, L)] = s_v
        stats_v[pl.ds(sb + 32, L)] = e_v
        stats_v[pl.ds(sb + 48, L)] = ptv
        cval_v[pl.ds(sb, L)] = trv[pl.ds(0, L)]
        cval_v[pl.ds(sb + 16, L)] = trv[pl.ds(16, L)]
        cval_v[pl.ds(sb + 32, L)] = trv[pl.ds(32, L)]
        cval_v[pl.ds(sb + 48, L)] = trv[pl.ds(48, L)]
        cidx_v[pl.ds(sb, L)] = tri[pl.ds(0, L)]
        cidx_v[pl.ds(sb + 16, L)] = tri[pl.ds(16, L)]
        cidx_v[pl.ds(sb + 32, L)] = tri[pl.ds(32, L)]
        cidx_v[pl.ds(sb + 48, L)] = tri[pl.ds(48, L)]

    def pair_step(g, _):
        do_row(2 * g, bufs[0], sems[0], sems[1], bufs[1])
        do_row(2 * g + 1, bufs[1], sems[1], sems[0], bufs[0])
        return 0

    lax.fori_loop(0, RPW // 2, pair_step, 0)

    pltpu.sync_copy(stats_v, stats_hbm.at[pl.ds(base * 64, RPW * 64)])
    pltpu.sync_copy(cval_v, cval_hbm.at[pl.ds(base * 64, RPW * 64)])
    pltpu.sync_copy(cidx_v, cidx_hbm.at[pl.ds(base * 64, RPW * 64)])


def _fin_body(stats_ref, cval_ref, cidx_ref, tgt_ref, out_ref):
    i = pl.program_id(0)
    R = stats_ref.shape[0]
    stats = stats_ref[...]
    m_v = stats[:, 0:16]
    sum_v = stats[:, 16:32]
    e_v = stats[:, 32:48]
    pt = jnp.max(stats[:, 48:64], axis=1)

    M = jnp.max(m_v, axis=1)
    S = jnp.sum(e_v * jnp.exp(m_v - M[:, None]), axis=1)
    lse = M + jnp.log(S)
    rowsum = jnp.sum(sum_v, axis=1)
    full = EPS * (rowsum - C * lse) + (HI - EPS) * (pt - lse)

    cval = cval_ref[...]
    cidx = cidx_ref[...]
    alive = jnp.ones(cval.shape, jnp.bool_)
    tv = []
    ti = []
    for _ in range(4):
        mv = jnp.where(alive, cval, -jnp.inf)
        cur = jnp.max(mv, axis=1)
        cand = mv == cur[:, None]
        curi = jnp.min(jnp.where(cand, cidx, C), axis=1)
        tv.append(cur)
        ti.append(curi)
        alive = alive & ~(cand & (cidx == curi[:, None]))

    rows = i * R + lax.broadcasted_iota(jnp.int32, (R,), 0)
    in0 = ti[0] == rows
    in1 = ti[1] == rows
    in2 = ti[2] == rows
    tgt = tgt_ref[:, 0]

    def term(v, idx):
        w = jnp.where(idx == tgt, HI, EPS)
        return w * (v - lse)

    sk0 = jnp.where(in0, term(tv[1], ti[1]), term(tv[0], ti[0]))
    sk1 = jnp.where(in0 | in1, term(tv[2], ti[2]), term(tv[1], ti[1]))
    sk2 = jnp.where(in0 | in1 | in2, term(tv[3], ti[3]), term(tv[2], ti[2]))
    skipped = sk0 + sk1 + sk2

    loss = -(full - skipped)
    part = jnp.reshape(jnp.sum(loss) * (1.0 / C), (1, 1))

    @pl.when(i == 0)
    def _():
        out_ref[...] = jnp.zeros((1, 1), jnp.float32)

    out_ref[...] += part


def _sc_call(preds_flat, targets):
    mesh = plsc.VectorSubcoreMesh(core_axis_name="c", subcore_axis_name="s",
                                  num_cores=NC, num_subcores=NS)
    f = functools.partial(
        pl.kernel,
        mesh=mesh,
        out_type=[
            jax.ShapeDtypeStruct((C * 64,), jnp.float32),
            jax.ShapeDtypeStruct((C * 64,), jnp.float32),
            jax.ShapeDtypeStruct((C * 64,), jnp.int32),
        ],
        scratch_types=[
            pltpu.VMEM((C,), jnp.float32),
            pltpu.VMEM((C,), jnp.float32),
            pltpu.VMEM((RPW,), jnp.int32),
            pltpu.VMEM((NGRP * L,), jnp.float32),
            pltpu.VMEM((64,), jnp.float32),
            pltpu.VMEM((64,), jnp.int32),
            pltpu.VMEM((RPW * 64,), jnp.float32),
            pltpu.VMEM((RPW * 64,), jnp.float32),
            pltpu.VMEM((RPW * 64,), jnp.int32),
            pltpu.SemaphoreType.DMA,
            pltpu.SemaphoreType.DMA,
        ],
        compiler_params=pltpu.CompilerParams(needs_layout_passes=False),
    )(_sc_body)
    return f(preds_flat, targets)


def kernel(preds, targets):
    preds_flat = preds.reshape(-1)
    tgt = targets.astype(jnp.int32)
    stats, cval, cidx = _sc_call(preds_flat, tgt)

    R = 512
    out = pl.pallas_call(
        _fin_body,
        grid=(C // R,),
        in_specs=[
            pl.BlockSpec((R, 64), lambda i: (i, 0)),
            pl.BlockSpec((R, 64), lambda i: (i, 0)),
            pl.BlockSpec((R, 64), lambda i: (i, 0)),
            pl.BlockSpec((R, 1), lambda i: (i, 0)),
        ],
        out_specs=pl.BlockSpec((1, 1), lambda i: (0, 0)),
        out_shape=jax.ShapeDtypeStruct((1, 1), jnp.float32),
    )(stats.reshape(C, 64), cval.reshape(C, 64), cidx.reshape(C, 64),
      tgt.reshape(C, 1))
    return out[0, 0]


# speedup vs baseline: 1.3484x; 1.2502x over previous
import functools

import jax
import jax.numpy as jnp
from jax import lax
from jax.experimental import pallas as pl
from jax.experimental.pallas import tpu as pltpu
from jax.experimental.pallas import tpu_sc as plsc

C = 4096
L = 16
NCHUNK = C // L
NC = 2
NS = 16
NW = NC * NS
RPW = C // NW
LABEL_SMOOTH = 0.1
EPS = LABEL_SMOOTH / (C - 1)
HI = 1.0 - LABEL_SMOOTH


GRP = 16
NGRP = 256 // GRP


def _sc_body(preds_hbm, tgt_hbm, stats_hbm, cval_hbm, cidx_hbm,
             rowbuf0, rowbuf1, tgtbuf, gbuf, trv, tri,
             stats_v, cval_v, cidx_v, sem0, sem1):
    wid = lax.axis_index("s") * NC + lax.axis_index("c")
    base = wid * RPW
    pltpu.sync_copy(tgt_hbm.at[pl.ds(base, RPW)], tgtbuf)

    iota = lax.iota(jnp.int32, L)
    zf = jnp.zeros((L,), jnp.float32)
    zi = jnp.zeros((L,), jnp.int32)
    ninf = jnp.full((L,), -jnp.inf, jnp.float32)

    bufs = (rowbuf0, rowbuf1)
    sems = (sem0, sem1)
    pltpu.async_copy(preds_hbm.at[base], rowbuf0, sem0)

    def do_row(j, rowbuf, sem, osem, obuf):
        row = base + j
        pltpu.make_async_copy(preds_hbm.at[base], rowbuf,
                              sem).wait()

        @pl.when(j + 1 < RPW)
        def _():
            pltpu.async_copy(preds_hbm.at[row + 1], obuf,
                             osem)

        def ga(g, carry):
            m_v, s0, s1, s2, s3 = carry

            def gi(t, c2):
                a0, a1, a2, a3, s0, s1, s2, s3 = c2
                b = (g * GRP + t * 4) * L
                v0 = rowbuf[pl.ds(b, L)]
                v1 = rowbuf[pl.ds(b + L, L)]
                v2 = rowbuf[pl.ds(b + 2 * L, L)]
                v3 = rowbuf[pl.ds(b + 3 * L, L)]
                return (jnp.maximum(a0, v0), jnp.maximum(a1, v1),
                        jnp.maximum(a2, v2), jnp.maximum(a3, v3),
                        s0 + v0, s1 + v1, s2 + v2, s3 + v3)

            a0, a1, a2, a3, s0, s1, s2, s3 = lax.fori_loop(
                0, GRP // 4, gi,
                (ninf, ninf, ninf, ninf, s0, s1, s2, s3), unroll=4)
            gm = jnp.maximum(jnp.maximum(a0, a1), jnp.maximum(a2, a3))
            gbuf[pl.ds(g * L, L)] = gm
            return (jnp.maximum(m_v, gm), s0, s1, s2, s3)

        m_v, s0, s1, s2, s3 = lax.fori_loop(
            0, NGRP, ga, (ninf, zf, zf, zf, zf))
        s_v = (s0 + s1) + (s2 + s3)

        mm = m_v
        for _ in range(3):
            t = jnp.max(mm)
            mm = jnp.where(mm == jnp.full((L,), t), ninf, mm)
        tau_v = jnp.full((L,), jnp.max(mm))

        trv[pl.ds(0, L)] = ninf
        trv[pl.ds(16, L)] = ninf
        trv[pl.ds(32, L)] = ninf
        trv[pl.ds(48, L)] = ninf
        tri[pl.ds(0, L)] = zi
        tri[pl.ds(16, L)] = zi
        tri[pl.ds(32, L)] = zi
        tri[pl.ds(48, L)] = zi

        def gb(g, _):
            gm = gbuf[pl.ds(g * L, L)]
            hit = jnp.any(gm >= tau_v)

            @pl.when(hit)
            def _():
                r0v = trv[pl.ds(0, L)]
                r1v = trv[pl.ds(16, L)]
                r2v = trv[pl.ds(32, L)]
                r3v = trv[pl.ds(48, L)]
                r0i = tri[pl.ds(0, L)]
                r1i = tri[pl.ds(16, L)]
                r2i = tri[pl.ds(32, L)]
                r3i = tri[pl.ds(48, L)]

                def ins(k, carry):
                    c0v, c1v, c2v, c3v, c0i, c1i, c2i, c3i = carry
                    c = g * GRP + k
                    v = rowbuf[pl.ds(c * L, L)]
                    cols = iota + c * L
                    w0 = v > c0v
                    w1 = v > c1v
                    w2 = v > c2v
                    w3 = v > c3v
                    n0v = jnp.where(w0, v, c0v)
                    n0i = jnp.where(w0, cols, c0i)
                    n1v = jnp.where(w0, c0v, jnp.where(w1, v, c1v))
                    n1i = jnp.where(w0, c0i, jnp.where(w1, cols, c1i))
                    n2v = jnp.where(w1, c1v, jnp.where(w2, v, c2v))
                    n2i = jnp.where(w1, c1i, jnp.where(w2, cols, c2i))
                    n3v = jnp.where(w2, c2v, jnp.where(w3, v, c3v))
                    n3i = jnp.where(w2, c2i, jnp.where(w3, cols, c3i))
                    return (n0v, n1v, n2v, n3v, n0i, n1i, n2i, n3i)

                r0v, r1v, r2v, r3v, r0i, r1i, r2i, r3i = lax.fori_loop(
                    0, GRP, ins,
                    (r0v, r1v, r2v, r3v, r0i, r1i, r2i, r3i), unroll=4)
                trv[pl.ds(0, L)] = r0v
                trv[pl.ds(16, L)] = r1v
                trv[pl.ds(32, L)] = r2v
                trv[pl.ds(48, L)] = r3v
                tri[pl.ds(0, L)] = r0i
                tri[pl.ds(16, L)] = r1i
                tri[pl.ds(32, L)] = r2i
                tri[pl.ds(48, L)] = r3i

            return 0

        lax.fori_loop(0, NGRP, gb, 0)

        def p2(t, c2):
            e0, e1, e2, e3 = c2
            b = t * 4 * L
            e0 = e0 + jnp.exp(rowbuf[pl.ds(b, L)] - m_v)
            e1 = e1 + jnp.exp(rowbuf[pl.ds(b + L, L)] - m_v)
            e2 = e2 + jnp.exp(rowbuf[pl.ds(b + 2 * L, L)] - m_v)
            e3 = e3 + jnp.exp(rowbuf[pl.ds(b + 3 * L, L)] - m_v)
            return (e0, e1, e2, e3)

        e0, e1, e2, e3 = lax.fori_loop(0, NCHUNK // 4, p2,
                                       (zf, zf, zf, zf), unroll=4)
        e_v = (e0 + e1) + (e2 + e3)

        tj = plsc.load_gather(tgtbuf, [jnp.full((L,), j, jnp.int32)])
        ptv = plsc.load_gather(rowbuf, [tj])

        sb = j * 64
        stats_v[pl.ds(sb, L)] = m_v
        stats_v[pl.ds(sb + 16, L)] = s_v
        stats_v[pl.ds(sb + 32, L)] = e_v
        stats_v[pl.ds(sb + 48, L)] = ptv
        cval_v[pl.ds(sb, L)] = trv[pl.ds(0, L)]
        cval_v[pl.ds(sb + 16, L)] = trv[pl.ds(16, L)]
        cval_v[pl.ds(sb + 32, L)] = trv[pl.ds(32, L)]
        cval_v[pl.ds(sb + 48, L)] = trv[pl.ds(48, L)]
        cidx_v[pl.ds(sb, L)] = tri[pl.ds(0, L)]
        cidx_v[pl.ds(sb + 16, L)] = tri[pl.ds(16, L)]
        cidx_v[pl.ds(sb + 32, L)] = tri[pl.ds(32, L)]
        cidx_v[pl.ds(sb + 48, L)] = tri[pl.ds(48, L)]

    def pair_step(g, _):
        do_row(2 * g, bufs[0], sems[0], sems[1], bufs[1])
        do_row(2 * g + 1, bufs[1], sems[1], sems[0], bufs[0])
        return 0

    lax.fori_loop(0, RPW // 2, pair_step, 0)

    pltpu.sync_copy(stats_v, stats_hbm.at[pl.ds(base * 64, RPW * 64)])
    pltpu.sync_copy(cval_v, cval_hbm.at[pl.ds(base * 64, RPW * 64)])
    pltpu.sync_copy(cidx_v, cidx_hbm.at[pl.ds(base * 64, RPW * 64)])


def _fin_body(stats_ref, cval_ref, cidx_ref, tgt_ref, out_ref):
    i = pl.program_id(0)
    R = stats_ref.shape[0]
    stats = stats_ref[...]
    m_v = stats[:, 0:16]
    sum_v = stats[:, 16:32]
    e_v = stats[:, 32:48]
    pt = jnp.max(stats[:, 48:64], axis=1)

    M = jnp.max(m_v, axis=1)
    S = jnp.sum(e_v * jnp.exp(m_v - M[:, None]), axis=1)
    lse = M + jnp.log(S)
    rowsum = jnp.sum(sum_v, axis=1)
    full = EPS * (rowsum - C * lse) + (HI - EPS) * (pt - lse)

    cval = cval_ref[...]
    cidx = cidx_ref[...]
    alive = jnp.ones(cval.shape, jnp.bool_)
    tv = []
    ti = []
    for _ in range(4):
        mv = jnp.where(alive, cval, -jnp.inf)
        cur = jnp.max(mv, axis=1)
        cand = mv == cur[:, None]
        curi = jnp.min(jnp.where(cand, cidx, C), axis=1)
        tv.append(cur)
        ti.append(curi)
        alive = alive & ~(cand & (cidx == curi[:, None]))

    rows = i * R + lax.broadcasted_iota(jnp.int32, (R,), 0)
    in0 = ti[0] == rows
    in1 = ti[1] == rows
    in2 = ti[2] == rows
    tgt = tgt_ref[:, 0]

    def term(v, idx):
        w = jnp.where(idx == tgt, HI, EPS)
        return w * (v - lse)

    sk0 = jnp.where(in0, term(tv[1], ti[1]), term(tv[0], ti[0]))
    sk1 = jnp.where(in0 | in1, term(tv[2], ti[2]), term(tv[1], ti[1]))
    sk2 = jnp.where(in0 | in1 | in2, term(tv[3], ti[3]), term(tv[2], ti[2]))
    skipped = sk0 + sk1 + sk2

    loss = -(full - skipped)
    part = jnp.reshape(jnp.sum(loss) * (1.0 / C), (1, 1))

    @pl.when(i == 0)
    def _():
        out_ref[...] = jnp.zeros((1, 1), jnp.float32)

    out_ref[...] += part


def _sc_call(preds2d, targets):
    mesh = plsc.VectorSubcoreMesh(core_axis_name="c", subcore_axis_name="s",
                                  num_cores=NC, num_subcores=NS)
    f = functools.partial(
        pl.kernel,
        mesh=mesh,
        out_type=[
            jax.ShapeDtypeStruct((C * 64,), jnp.float32),
            jax.ShapeDtypeStruct((C * 64,), jnp.float32),
            jax.ShapeDtypeStruct((C * 64,), jnp.int32),
        ],
        scratch_types=[
            pltpu.VMEM((C,), jnp.float32),
            pltpu.VMEM((C,), jnp.float32),
            pltpu.VMEM((RPW,), jnp.int32),
            pltpu.VMEM((NGRP * L,), jnp.float32),
            pltpu.VMEM((64,), jnp.float32),
            pltpu.VMEM((64,), jnp.int32),
            pltpu.VMEM((RPW * 64,), jnp.float32),
            pltpu.VMEM((RPW * 64,), jnp.float32),
            pltpu.VMEM((RPW * 64,), jnp.int32),
            pltpu.SemaphoreType.DMA,
            pltpu.SemaphoreType.DMA,
        ],
        compiler_params=pltpu.CompilerParams(needs_layout_passes=False),
    )(_sc_body)
    return f(preds2d, targets)


def kernel(preds, targets):
    tgt = targets.astype(jnp.int32)
    stats, cval, cidx = _sc_call(preds, tgt)

    R = 512
    out = pl.pallas_call(
        _fin_body,
        grid=(C // R,),
        in_specs=[
            pl.BlockSpec((R, 64), lambda i: (i, 0)),
            pl.BlockSpec((R, 64), lambda i: (i, 0)),
            pl.BlockSpec((R, 64), lambda i: (i, 0)),
            pl.BlockSpec((R, 1), lambda i: (i, 0)),
        ],
        out_specs=pl.BlockSpec((1, 1), lambda i: (0, 0)),
        out_shape=jax.ShapeDtypeStruct((1, 1), jnp.float32),
    )(stats.reshape(C, 64), cval.reshape(C, 64), cidx.reshape(C, 64),
      tgt.reshape(C, 1))
    return out[0, 0]


# speedup vs baseline: 1.5322x; 1.1363x over previous
import functools

import jax
import jax.numpy as jnp
from jax import lax
from jax.experimental import pallas as pl
from jax.experimental.pallas import tpu as pltpu
from jax.experimental.pallas import tpu_sc as plsc

C = 4096
L = 16
NCHUNK = C // L
NC = 2
NS = 16
NW = NC * NS
RPW = C // NW
LABEL_SMOOTH = 0.1
EPS = LABEL_SMOOTH / (C - 1)
HI = 1.0 - LABEL_SMOOTH


GRP = 16
NGRP = 256 // GRP


def _sc_body(preds_hbm, tgt_hbm, stats_hbm, cval_hbm, cidx_hbm,
             rowbuf0, rowbuf1, tgtbuf, gbuf, trv, tri,
             stats_v, cval_v, cidx_v, sem0, sem1):
    wid = lax.axis_index("s") * NC + lax.axis_index("c")
    base = wid * RPW
    pltpu.sync_copy(tgt_hbm.at[pl.ds(base, RPW)], tgtbuf)

    iota = lax.iota(jnp.int32, L)
    zf = jnp.zeros((L,), jnp.float32)
    zi = jnp.zeros((L,), jnp.int32)
    ninf = jnp.full((L,), -jnp.inf, jnp.float32)

    bufs = (rowbuf0, rowbuf1)
    sems = (sem0, sem1)
    pltpu.async_copy(preds_hbm.at[base], rowbuf0, sem0)

    def do_row(j, rowbuf, sem, osem, obuf):
        row = base + j
        pltpu.make_async_copy(preds_hbm.at[base], rowbuf,
                              sem).wait()

        @pl.when(j + 1 < RPW)
        def _():
            pltpu.async_copy(preds_hbm.at[row + 1], obuf,
                             osem)

        def ga(g, carry):
            m_v = carry

            def gi(t, c2):
                a0, a1, a2, a3 = c2
                b = (g * GRP + t * 4) * L
                v0 = rowbuf[pl.ds(b, L)]
                v1 = rowbuf[pl.ds(b + L, L)]
                v2 = rowbuf[pl.ds(b + 2 * L, L)]
                v3 = rowbuf[pl.ds(b + 3 * L, L)]
                return (jnp.maximum(a0, v0), jnp.maximum(a1, v1),
                        jnp.maximum(a2, v2), jnp.maximum(a3, v3))

            a0, a1, a2, a3 = lax.fori_loop(
                0, GRP // 4, gi, (ninf, ninf, ninf, ninf), unroll=4)
            gm = jnp.maximum(jnp.maximum(a0, a1), jnp.maximum(a2, a3))
            gbuf[pl.ds(g * L, L)] = gm
            return jnp.maximum(m_v, gm)

        m_v = lax.fori_loop(0, NGRP, ga, ninf)

        mm = m_v
        for _ in range(3):
            t = jnp.max(mm)
            mm = jnp.where(mm == jnp.full((L,), t), ninf, mm)
        tau_v = jnp.full((L,), jnp.max(mm))

        trv[pl.ds(0, L)] = ninf
        trv[pl.ds(16, L)] = ninf
        trv[pl.ds(32, L)] = ninf
        trv[pl.ds(48, L)] = ninf
        tri[pl.ds(0, L)] = zi
        tri[pl.ds(16, L)] = zi
        tri[pl.ds(32, L)] = zi
        tri[pl.ds(48, L)] = zi

        def gb(g, _):
            gm = gbuf[pl.ds(g * L, L)]
            hit = jnp.any(gm >= tau_v)

            @pl.when(hit)
            def _():
                r0v = trv[pl.ds(0, L)]
                r1v = trv[pl.ds(16, L)]
                r2v = trv[pl.ds(32, L)]
                r3v = trv[pl.ds(48, L)]
                r0i = tri[pl.ds(0, L)]
                r1i = tri[pl.ds(16, L)]
                r2i = tri[pl.ds(32, L)]
                r3i = tri[pl.ds(48, L)]

                def ins(k, carry):
                    c0v, c1v, c2v, c3v, c0i, c1i, c2i, c3i = carry
                    c = g * GRP + k
                    v = rowbuf[pl.ds(c * L, L)]
                    cols = iota + c * L
                    w0 = v > c0v
                    w1 = v > c1v
                    w2 = v > c2v
                    w3 = v > c3v
                    n0v = jnp.where(w0, v, c0v)
                    n0i = jnp.where(w0, cols, c0i)
                    n1v = jnp.where(w0, c0v, jnp.where(w1, v, c1v))
                    n1i = jnp.where(w0, c0i, jnp.where(w1, cols, c1i))
                    n2v = jnp.where(w1, c1v, jnp.where(w2, v, c2v))
                    n2i = jnp.where(w1, c1i, jnp.where(w2, cols, c2i))
                    n3v = jnp.where(w2, c2v, jnp.where(w3, v, c3v))
                    n3i = jnp.where(w2, c2i, jnp.where(w3, cols, c3i))
                    return (n0v, n1v, n2v, n3v, n0i, n1i, n2i, n3i)

                r0v, r1v, r2v, r3v, r0i, r1i, r2i, r3i = lax.fori_loop(
                    0, GRP, ins,
                    (r0v, r1v, r2v, r3v, r0i, r1i, r2i, r3i), unroll=4)
                trv[pl.ds(0, L)] = r0v
                trv[pl.ds(16, L)] = r1v
                trv[pl.ds(32, L)] = r2v
                trv[pl.ds(48, L)] = r3v
                tri[pl.ds(0, L)] = r0i
                tri[pl.ds(16, L)] = r1i
                tri[pl.ds(32, L)] = r2i
                tri[pl.ds(48, L)] = r3i

            return 0

        lax.fori_loop(0, NGRP, gb, 0)

        tj = plsc.load_gather(tgtbuf, [jnp.full((L,), j, jnp.int32)])
        ptv = plsc.load_gather(rowbuf, [tj])

        stats_v[pl.ds(j * L, L)] = ptv
        sb = j * 64
        cval_v[pl.ds(sb, L)] = trv[pl.ds(0, L)]
        cval_v[pl.ds(sb + 16, L)] = trv[pl.ds(16, L)]
        cval_v[pl.ds(sb + 32, L)] = trv[pl.ds(32, L)]
        cval_v[pl.ds(sb + 48, L)] = trv[pl.ds(48, L)]
        cidx_v[pl.ds(sb, L)] = tri[pl.ds(0, L)]
        cidx_v[pl.ds(sb + 16, L)] = tri[pl.ds(16, L)]
        cidx_v[pl.ds(sb + 32, L)] = tri[pl.ds(32, L)]
        cidx_v[pl.ds(sb + 48, L)] = tri[pl.ds(48, L)]

    def pair_step(g, _):
        do_row(2 * g, bufs[0], sems[0], sems[1], bufs[1])
        do_row(2 * g + 1, bufs[1], sems[1], sems[0], bufs[0])
        return 0

    lax.fori_loop(0, RPW // 2, pair_step, 0)

    pltpu.sync_copy(stats_v, stats_hbm.at[pl.ds(base * L, RPW * L)])
    pltpu.sync_copy(cval_v, cval_hbm.at[pl.ds(base * 64, RPW * 64)])
    pltpu.sync_copy(cidx_v, cidx_hbm.at[pl.ds(base * 64, RPW * 64)])


def _dense_body(preds_ref, out_ref):
    x = preds_ref[...]
    M = jnp.max(x, axis=1)
    S = jnp.sum(jnp.exp(x - M[:, None]), axis=1)
    rs = jnp.sum(x, axis=1)
    z = jnp.zeros_like(M)
    out_ref[...] = jnp.stack([M, S, rs, z, z, z, z, z], axis=0)


def _fin_body(dstats_ref, pt_ref, cval_ref, cidx_ref, tgt_ref, out_ref):
    i = pl.program_id(0)
    R = pt_ref.shape[0]
    dstats = dstats_ref[...]
    M = dstats[0, :]
    S = dstats[1, :]
    rowsum = dstats[2, :]
    pt = jnp.max(pt_ref[...], axis=1)

    lse = M + jnp.log(S)
    full = EPS * (rowsum - C * lse) + (HI - EPS) * (pt - lse)

    cval = cval_ref[...]
    cidx = cidx_ref[...]
    alive = jnp.ones(cval.shape, jnp.bool_)
    tv = []
    ti = []
    for _ in range(4):
        mv = jnp.where(alive, cval, -jnp.inf)
        cur = jnp.max(mv, axis=1)
        cand = mv == cur[:, None]
        curi = jnp.min(jnp.where(cand, cidx, C), axis=1)
        tv.append(cur)
        ti.append(curi)
        alive = alive & ~(cand & (cidx == curi[:, None]))

    rows = i * R + lax.broadcasted_iota(jnp.int32, (R,), 0)
    in0 = ti[0] == rows
    in1 = ti[1] == rows
    in2 = ti[2] == rows
    tgt = tgt_ref[:, 0]

    def term(v, idx):
        w = jnp.where(idx == tgt, HI, EPS)
        return w * (v - lse)

    sk0 = jnp.where(in0, term(tv[1], ti[1]), term(tv[0], ti[0]))
    sk1 = jnp.where(in0 | in1, term(tv[2], ti[2]), term(tv[1], ti[1]))
    sk2 = jnp.where(in0 | in1 | in2, term(tv[3], ti[3]), term(tv[2], ti[2]))
    skipped = sk0 + sk1 + sk2

    loss = -(full - skipped)
    part = jnp.reshape(jnp.sum(loss) * (1.0 / C), (1, 1))

    @pl.when(i == 0)
    def _():
        out_ref[...] = jnp.zeros((1, 1), jnp.float32)

    out_ref[...] += part


def _sc_call(preds2d, targets):
    mesh = plsc.VectorSubcoreMesh(core_axis_name="c", subcore_axis_name="s",
                                  num_cores=NC, num_subcores=NS)
    f = functools.partial(
        pl.kernel,
        mesh=mesh,
        out_type=[
            jax.ShapeDtypeStruct((C * L,), jnp.float32),
            jax.ShapeDtypeStruct((C * 64,), jnp.float32),
            jax.ShapeDtypeStruct((C * 64,), jnp.int32),
        ],
        scratch_types=[
            pltpu.VMEM((C,), jnp.float32),
            pltpu.VMEM((C,), jnp.float32),
            pltpu.VMEM((RPW,), jnp.int32),
            pltpu.VMEM((NGRP * L,), jnp.float32),
            pltpu.VMEM((64,), jnp.float32),
            pltpu.VMEM((64,), jnp.int32),
            pltpu.VMEM((RPW * L,), jnp.float32),
            pltpu.VMEM((RPW * 64,), jnp.float32),
            pltpu.VMEM((RPW * 64,), jnp.int32),
            pltpu.SemaphoreType.DMA,
            pltpu.SemaphoreType.DMA,
        ],
        compiler_params=pltpu.CompilerParams(needs_layout_passes=False),
    )(_sc_body)
    return f(preds2d, targets)


def kernel(preds, targets):
    tgt = targets.astype(jnp.int32)
    ptv, cval, cidx = _sc_call(preds, tgt)

    RD = 256
    dstats = pl.pallas_call(
        _dense_body,
        grid=(C // RD,),
        in_specs=[pl.BlockSpec((RD, C), lambda i: (i, 0))],
        out_specs=pl.BlockSpec((8, RD), lambda i: (0, i)),
        out_shape=jax.ShapeDtypeStruct((8, C), jnp.float32),
    )(preds)

    R = 512
    out = pl.pallas_call(
        _fin_body,
        grid=(C // R,),
        in_specs=[
            pl.BlockSpec((8, R), lambda i: (0, i)),
            pl.BlockSpec((R, 16), lambda i: (i, 0)),
            pl.BlockSpec((R, 64), lambda i: (i, 0)),
            pl.BlockSpec((R, 64), lambda i: (i, 0)),
            pl.BlockSpec((R, 1), lambda i: (i, 0)),
        ],
        out_specs=pl.BlockSpec((1, 1), lambda i: (0, 0)),
        out_shape=jax.ShapeDtypeStruct((1, 1), jnp.float32),
    )(dstats, ptv.reshape(C, 16), cval.reshape(C, 64), cidx.reshape(C, 64),
      tgt.reshape(C, 1))
    return out[0, 0]


# speedup vs baseline: 1.6461x; 1.0744x over previous
import functools

import jax
import jax.numpy as jnp
from jax import lax
from jax.experimental import pallas as pl
from jax.experimental.pallas import tpu as pltpu
from jax.experimental.pallas import tpu_sc as plsc

C = 4096
L = 16
NCHUNK = C // L
NC = 2
NS = 16
NW = NC * NS
RPW = C // NW
LABEL_SMOOTH = 0.1
EPS = LABEL_SMOOTH / (C - 1)
HI = 1.0 - LABEL_SMOOTH


GRP = 16
NGRP = 256 // GRP


def _sc_body(preds_hbm, tgt_hbm, stats_hbm, cval_hbm, cidx_hbm,
             ra0, ra1, rb0, rb1, tgtbuf, gbufa, gbufb, trv, tri,
             stats_v, cval_v, cidx_v, sema, semb):
    wid = lax.axis_index("s") * NC + lax.axis_index("c")
    base = wid * RPW
    pltpu.sync_copy(tgt_hbm.at[pl.ds(base, RPW)], tgtbuf)

    iota = lax.iota(jnp.int32, L)
    zi = jnp.zeros((L,), jnp.int32)
    ninf = jnp.full((L,), -jnp.inf, jnp.float32)

    pltpu.async_copy(preds_hbm.at[base], ra0, sema)
    pltpu.async_copy(preds_hbm.at[base + 1], ra1, sema)

    def do_pair(j, pb0, pb1, psem, ob0, ob1, osem, gba, gbb):
        row = base + j
        pltpu.make_async_copy(preds_hbm.at[base], pb0, psem).wait()
        pltpu.make_async_copy(preds_hbm.at[base], pb1, psem).wait()

        @pl.when(j + 2 < RPW)
        def _():
            pltpu.async_copy(preds_hbm.at[row + 2], ob0, osem)
            pltpu.async_copy(preds_hbm.at[row + 3], ob1, osem)

        def ga(g, carry):
            ma, mb = carry

            def gi(t, c2):
                a0, a1, b0, b1 = c2
                o = (g * GRP + t * 2) * L
                va0 = pb0[pl.ds(o, L)]
                va1 = pb0[pl.ds(o + L, L)]
                vb0 = pb1[pl.ds(o, L)]
                vb1 = pb1[pl.ds(o + L, L)]
                return (jnp.maximum(a0, va0), jnp.maximum(a1, va1),
                        jnp.maximum(b0, vb0), jnp.maximum(b1, vb1))

            a0, a1, b0, b1 = lax.fori_loop(
                0, GRP // 2, gi, (ninf, ninf, ninf, ninf), unroll=4)
            gma = jnp.maximum(a0, a1)
            gmb = jnp.maximum(b0, b1)
            gba[pl.ds(g * L, L)] = gma
            gbb[pl.ds(g * L, L)] = gmb
            return (jnp.maximum(ma, gma), jnp.maximum(mb, gmb))

        m_a, m_b = lax.fori_loop(0, NGRP, ga, (ninf, ninf))

        bcast3 = jnp.full((L,), 3, jnp.int32)
        sa, _ = plsc.sort_key_val(m_a, m_a, descending=True)
        tau_a = sa.at[bcast3].get(mode="promise_in_bounds")
        sb_, _ = plsc.sort_key_val(m_b, m_b, descending=True)
        tau_b = sb_.at[bcast3].get(mode="promise_in_bounds")

        for off in (0, 16, 32, 48):
            trv[pl.ds(off, L)] = ninf
            trv[pl.ds(64 + off, L)] = ninf
            tri[pl.ds(off, L)] = zi
            tri[pl.ds(64 + off, L)] = zi

        def mk_gb(gbuf, rowbuf, tau_v, toff):
            def gb(g, _):
                gm = gbuf[pl.ds(g * L, L)]
                hit = jnp.any(gm >= tau_v)

                @pl.when(hit)
                def _():
                    r0v = trv[pl.ds(toff, L)]
                    r1v = trv[pl.ds(toff + 16, L)]
                    r2v = trv[pl.ds(toff + 32, L)]
                    r3v = trv[pl.ds(toff + 48, L)]
                    r0i = tri[pl.ds(toff, L)]
                    r1i = tri[pl.ds(toff + 16, L)]
                    r2i = tri[pl.ds(toff + 32, L)]
                    r3i = tri[pl.ds(toff + 48, L)]

                    def ins(k, carry):
                        c0v, c1v, c2v, c3v, c0i, c1i, c2i, c3i = carry
                        c = g * GRP + k
                        v = rowbuf[pl.ds(c * L, L)]
                        cols = iota + c * L
                        w0 = v > c0v
                        w1 = v > c1v
                        w2 = v > c2v
                        w3 = v > c3v
                        n0v = jnp.where(w0, v, c0v)
                        n0i = jnp.where(w0, cols, c0i)
                        n1v = jnp.where(w0, c0v, jnp.where(w1, v, c1v))
                        n1i = jnp.where(w0, c0i, jnp.where(w1, cols, c1i))
                        n2v = jnp.where(w1, c1v, jnp.where(w2, v, c2v))
                        n2i = jnp.where(w1, c1i, jnp.where(w2, cols, c2i))
                        n3v = jnp.where(w2, c2v, jnp.where(w3, v, c3v))
                        n3i = jnp.where(w2, c2i, jnp.where(w3, cols, c3i))
                        return (n0v, n1v, n2v, n3v, n0i, n1i, n2i, n3i)

                    r0v, r1v, r2v, r3v, r0i, r1i, r2i, r3i = lax.fori_loop(
                        0, GRP, ins,
                        (r0v, r1v, r2v, r3v, r0i, r1i, r2i, r3i), unroll=4)
                    trv[pl.ds(toff, L)] = r0v
                    trv[pl.ds(toff + 16, L)] = r1v
                    trv[pl.ds(toff + 32, L)] = r2v
                    trv[pl.ds(toff + 48, L)] = r3v
                    tri[pl.ds(toff, L)] = r0i
                    tri[pl.ds(toff + 16, L)] = r1i
                    tri[pl.ds(toff + 32, L)] = r2i
                    tri[pl.ds(toff + 48, L)] = r3i

                return 0
            return gb

        lax.fori_loop(0, NGRP, mk_gb(gba, pb0, tau_a, 0), 0)
        lax.fori_loop(0, NGRP, mk_gb(gbb, pb1, tau_b, 64), 0)

        ta = plsc.load_gather(tgtbuf, [jnp.full((L,), j, jnp.int32)])
        pa = plsc.load_gather(pb0, [ta])
        tb = plsc.load_gather(tgtbuf, [jnp.full((L,), j + 1, jnp.int32)])
        pb = plsc.load_gather(pb1, [tb])
        stats_v[pl.ds(j * L, L)] = pa
        stats_v[pl.ds((j + 1) * L, L)] = pb

        for r, toff in ((j, 0), (j + 1, 64)):
            sb = r * 64
            cval_v[pl.ds(sb, L)] = trv[pl.ds(toff, L)]
            cval_v[pl.ds(sb + 16, L)] = trv[pl.ds(toff + 16, L)]
            cval_v[pl.ds(sb + 32, L)] = trv[pl.ds(toff + 32, L)]
            cval_v[pl.ds(sb + 48, L)] = trv[pl.ds(toff + 48, L)]
            cidx_v[pl.ds(sb, L)] = tri[pl.ds(toff, L)]
            cidx_v[pl.ds(sb + 16, L)] = tri[pl.ds(toff + 16, L)]
            cidx_v[pl.ds(sb + 32, L)] = tri[pl.ds(toff + 32, L)]
            cidx_v[pl.ds(sb + 48, L)] = tri[pl.ds(toff + 48, L)]

    def quad_step(q, _):
        do_pair(4 * q, ra0, ra1, sema, rb0, rb1, semb, gbufa, gbufb)
        do_pair(4 * q + 2, rb0, rb1, semb, ra0, ra1, sema, gbufa, gbufb)
        return 0

    lax.fori_loop(0, RPW // 4, quad_step, 0)

    pltpu.sync_copy(stats_v, stats_hbm.at[pl.ds(base * L, RPW * L)])
    pltpu.sync_copy(cval_v, cval_hbm.at[pl.ds(base * 64, RPW * 64)])
    pltpu.sync_copy(cidx_v, cidx_hbm.at[pl.ds(base * 64, RPW * 64)])


def _dense_body(preds_ref, out_ref):
    x = preds_ref[...]
    M = jnp.max(x, axis=1)
    S = jnp.sum(jnp.exp(x - M[:, None]), axis=1)
    rs = jnp.sum(x, axis=1)
    z = jnp.zeros_like(M)
    out_ref[...] = jnp.stack([M, S, rs, z, z, z, z, z], axis=0)


def _fin_body(dstats_ref, pt_ref, cval_ref, cidx_ref, tgt_ref, out_ref):
    i = pl.program_id(0)
    R = pt_ref.shape[0]
    dstats = dstats_ref[...]
    M = dstats[0, :]
    S = dstats[1, :]
    rowsum = dstats[2, :]
    pt = jnp.max(pt_ref[...], axis=1)

    lse = M + jnp.log(S)
    full = EPS * (rowsum - C * lse) + (HI - EPS) * (pt - lse)

    cval = cval_ref[...]
    cidx = cidx_ref[...]
    alive = jnp.ones(cval.shape, jnp.bool_)
    tv = []
    ti = []
    for _ in range(4):
        mv = jnp.where(alive, cval, -jnp.inf)
        cur = jnp.max(mv, axis=1)
        cand = mv == cur[:, None]
        curi = jnp.min(jnp.where(cand, cidx, C), axis=1)
        tv.append(cur)
        ti.append(curi)
        alive = alive & ~(cand & (cidx == curi[:, None]))

    rows = i * R + lax.broadcasted_iota(jnp.int32, (R,), 0)
    in0 = ti[0] == rows
    in1 = ti[1] == rows
    in2 = ti[2] == rows
    tgt = tgt_ref[:, 0]

    def term(v, idx):
        w = jnp.where(idx == tgt, HI, EPS)
        return w * (v - lse)

    sk0 = jnp.where(in0, term(tv[1], ti[1]), term(tv[0], ti[0]))
    sk1 = jnp.where(in0 | in1, term(tv[2], ti[2]), term(tv[1], ti[1]))
    sk2 = jnp.where(in0 | in1 | in2, term(tv[3], ti[3]), term(tv[2], ti[2]))
    skipped = sk0 + sk1 + sk2

    loss = -(full - skipped)
    part = jnp.reshape(jnp.sum(loss) * (1.0 / C), (1, 1))

    @pl.when(i == 0)
    def _():
        out_ref[...] = jnp.zeros((1, 1), jnp.float32)

    out_ref[...] += part


def _sc_call(preds2d, targets):
    mesh = plsc.VectorSubcoreMesh(core_axis_name="c", subcore_axis_name="s",
                                  num_cores=NC, num_subcores=NS)
    f = functools.partial(
        pl.kernel,
        mesh=mesh,
        out_type=[
            jax.ShapeDtypeStruct((C * L,), jnp.float32),
            jax.ShapeDtypeStruct((C * 64,), jnp.float32),
            jax.ShapeDtypeStruct((C * 64,), jnp.int32),
        ],
        scratch_types=[
            pltpu.VMEM((C,), jnp.float32),
            pltpu.VMEM((C,), jnp.float32),
            pltpu.VMEM((C,), jnp.float32),
            pltpu.VMEM((C,), jnp.float32),
            pltpu.VMEM((RPW,), jnp.int32),
            pltpu.VMEM((NGRP * L,), jnp.float32),
            pltpu.VMEM((NGRP * L,), jnp.float32),
            pltpu.VMEM((128,), jnp.float32),
            pltpu.VMEM((128,), jnp.int32),
            pltpu.VMEM((RPW * L,), jnp.float32),
            pltpu.VMEM((RPW * 64,), jnp.float32),
            pltpu.VMEM((RPW * 64,), jnp.int32),
            pltpu.SemaphoreType.DMA,
            pltpu.SemaphoreType.DMA,
        ],
        compiler_params=pltpu.CompilerParams(needs_layout_passes=False),
    )(_sc_body)
    return f(preds2d, targets)


def kernel(preds, targets):
    tgt = targets.astype(jnp.int32)
    ptv, cval, cidx = _sc_call(preds, tgt)

    RD = 256
    dstats = pl.pallas_call(
        _dense_body,
        grid=(C // RD,),
        in_specs=[pl.BlockSpec((RD, C), lambda i: (i, 0))],
        out_specs=pl.BlockSpec((8, RD), lambda i: (0, i)),
        out_shape=jax.ShapeDtypeStruct((8, C), jnp.float32),
    )(preds)

    R = 512
    out = pl.pallas_call(
        _fin_body,
        grid=(C // R,),
        in_specs=[
            pl.BlockSpec((8, R), lambda i: (0, i)),
            pl.BlockSpec((R, 16), lambda i: (i, 0)),
            pl.BlockSpec((R, 64), lambda i: (i, 0)),
            pl.BlockSpec((R, 64), lambda i: (i, 0)),
            pl.BlockSpec((R, 1), lambda i: (i, 0)),
        ],
        out_specs=pl.BlockSpec((1, 1), lambda i: (0, 0)),
        out_shape=jax.ShapeDtypeStruct((1, 1), jnp.float32),
    )(dstats, ptv.reshape(C, 16), cval.reshape(C, 64), cidx.reshape(C, 64),
      tgt.reshape(C, 1))
    return out[0, 0]
